# Initial kernel scaffold; baseline (speedup 1.0000x reference)
import jax
import jax.numpy as jnp
from jax import lax
from jax.experimental import pallas as pl
from jax.experimental.pallas import tpu as pltpu
from jax.experimental.pallas import tpu_sc as plsc

B, L, D = 1024, 200, 128
N = B * L
NW = 32
TPW = N // NW
CHUNK = 128
NCHUNK = TPW // CHUNK
GROUPS = CHUNK // 16
EPS = 1e-12


def _rsqrt(x):
    i = lax.bitcast_convert_type(x, jnp.int32)
    i = jnp.int32(0x5F3759DF) - lax.shift_right_logical(i, 1)
    y = lax.bitcast_convert_type(i, jnp.float32)
    for _ in range(3):
        y = y * (1.5 - 0.5 * x * y * y)
    return y


def _body(ids_hbm, tt_hbm, tab_hbm, pos_hbm, par_hbm, out_hbm,
          pos_v, par_v, idx_v, tt_v, chunk_v, x_v, outb_v, sem):
    wid = lax.axis_index("s") * 2 + lax.axis_index("c")
    pltpu.sync_copy(pos_hbm, pos_v)
    pltpu.sync_copy(par_hbm, par_v)
    pltpu.sync_copy(ids_hbm.at[wid], idx_v)
    pltpu.sync_copy(tt_hbm.at[wid], tt_v)

    iota = lax.iota(jnp.int32, 16)

    def chunk_body(c, carry):
        pltpu.async_copy(tab_hbm.at[idx_v.at[c]], chunk_v, sem).wait()
        tok0 = wid * TPW + c * CHUNK

        def group_body(g, gcarry):
            rows = g * 16 + iota
            ttf = tt_v[pl.ds(c * CHUNK + g * 16, 16)].astype(jnp.float32)
            p0 = lax.rem(tok0 + g * 16, L)
            pv = p0 + iota
            pv = jnp.where(pv >= L, pv - L, pv)
            acc = jnp.zeros((16,), jnp.float32)
            acc2 = jnp.zeros((16,), jnp.float32)
            for d in range(D):
                cold = jnp.full((16,), d, jnp.int32)
                xt = plsc.load_gather(chunk_v, [rows, cold])
                xp = plsc.load_gather(pos_v, [pv, cold])
                s0 = par_v[d]
                s1 = par_v[D + d]
                x = xt + xp + (ttf * (s1 - s0) + s0)
                acc = acc + x
                acc2 = acc2 + x * x
                x_v[d] = x
            mean = acc * (1.0 / D)
            var = acc2 * (1.0 / D) - mean * mean
            r = _rsqrt(var + EPS)
            for d in range(D):
                cold = jnp.full((16,), d, jnp.int32)
                o = (x_v[d] - mean) * (r * par_v[2 * D + d]) + par_v[3 * D + d]
                plsc.store_scatter(outb_v, [rows, cold], o)
            return gcarry

        lax.fori_loop(0, GROUPS, group_body, 0)
        pltpu.sync_copy(outb_v, out_hbm.at[pl.ds(tok0, CHUNK)])
        return carry

    lax.fori_loop(0, NCHUNK, chunk_body, 0)


def kernel(input_ids, token_type_ids, token_table, position_table,
           segment_table, gamma, beta):
    ids = input_ids.reshape(N).astype(jnp.int32).reshape(NW, NCHUNK, CHUNK)
    tt = token_type_ids.reshape(N).astype(jnp.int32).reshape(NW, TPW)
    pos = position_table[:L]
    par = jnp.concatenate([segment_table.reshape(-1), gamma, beta])
    mesh = plsc.VectorSubcoreMesh(core_axis_name="c", subcore_axis_name="s")
    out = pl.kernel(
        _body,
        out_type=jax.ShapeDtypeStruct((N, D), jnp.float32),
        mesh=mesh,
        scratch_types=[
            pltpu.VMEM((L, D), jnp.float32),
            pltpu.VMEM((4 * D,), jnp.float32),
            pltpu.VMEM((NCHUNK, CHUNK), jnp.int32),
            pltpu.VMEM((TPW,), jnp.int32),
            pltpu.VMEM((CHUNK, D), jnp.float32),
            pltpu.VMEM((D, 16), jnp.float32),
            pltpu.VMEM((CHUNK, D), jnp.float32),
            pltpu.SemaphoreType.DMA,
        ],
    )(ids, tt, token_table, pos, par)
    return out.reshape(B, L, D)


# speedup vs baseline: 4.5439x; 4.5439x over previous
import jax
import jax.numpy as jnp
from jax import lax
from jax.experimental import pallas as pl
from jax.experimental.pallas import tpu as pltpu
from jax.experimental.pallas import tpu_sc as plsc

B, L, D = 1024, 200, 128
N = B * L
NW = 32
TPW = N // NW
CHUNK = 128
NCHUNK = TPW // CHUNK
GROUPS = CHUNK // 16
NV = D // 16
EPS = 1e-12


def _rsqrt(x):
    i = lax.bitcast_convert_type(x, jnp.int32)
    i = jnp.int32(0x5F3759DF) - lax.shift_right_logical(i, 1)
    y = lax.bitcast_convert_type(i, jnp.float32)
    for _ in range(3):
        y = y * (1.5 - 0.5 * x * y * y)
    return y


def _body(ids_hbm, ps_idx_hbm, tab_hbm, ps_hbm, gb_hbm, out_hbm,
          ps_v, gb_v, idx_v, psi_v, chunk_v, outb_v, sem):
    wid = lax.axis_index("s") * 2 + lax.axis_index("c")
    pltpu.sync_copy(ps_hbm, ps_v)
    pltpu.sync_copy(gb_hbm, gb_v)
    pltpu.sync_copy(ids_hbm.at[wid], idx_v)
    pltpu.sync_copy(ps_idx_hbm.at[wid], psi_v)

    def chunk_body(c, carry):
        pltpu.async_copy(tab_hbm.at[idx_v.at[c]], chunk_v, sem).wait()
        tok0 = wid * TPW + c * CHUNK

        def group_body(g, gcarry):
            gv = [gb_v[pl.ds(16 * i, 16)] for i in range(NV)]
            bv = [gb_v[pl.ds(D + 16 * i, 16)] for i in range(NV)]
            pr16 = psi_v[pl.ds(c * CHUNK + g * 16, 16)]
            for t in range(16):
                j = g * 16 + t
                pbase = pr16[t] * D
                x = [chunk_v[j, pl.ds(16 * i, 16)]
                     + ps_v[pl.ds(pbase + 16 * i, 16)] for i in range(NV)]
                s1 = x[0]
                for i in range(1, NV):
                    s1 = s1 + x[i]
                s2 = x[0] * x[0]
                for i in range(1, NV):
                    s2 = s2 + x[i] * x[i]
                tot1 = jnp.sum(s1)
                tot2 = jnp.sum(s2)
                mean = tot1 * (1.0 / D)
                var = tot2 * (1.0 / D) - mean * mean
                r = _rsqrt(var + EPS)
                for i in range(NV):
                    o = (x[i] - mean) * r * gv[i] + bv[i]
                    outb_v[j, pl.ds(16 * i, 16)] = o
            return gcarry

        lax.fori_loop(0, GROUPS, group_body, 0)
        pltpu.sync_copy(outb_v, out_hbm.at[pl.ds(tok0, CHUNK)])
        return carry

    lax.fori_loop(0, NCHUNK, chunk_body, 0)


def kernel(input_ids, token_type_ids, token_table, position_table,
           segment_table, gamma, beta):
    ids = input_ids.reshape(N).astype(jnp.int32).reshape(NW, NCHUNK, CHUNK)
    ps_idx = ((jnp.arange(N, dtype=jnp.int32) % L) * 2
              + token_type_ids.reshape(N).astype(jnp.int32)).reshape(NW, TPW)
    ps = (position_table[:L, None, :] + segment_table[None, :, :]).reshape(
        2 * L * D)
    gb = jnp.concatenate([gamma, beta])
    mesh = plsc.VectorSubcoreMesh(core_axis_name="c", subcore_axis_name="s")
    out = pl.kernel(
        _body,
        out_type=jax.ShapeDtypeStruct((N, D), jnp.float32),
        mesh=mesh,
        compiler_params=pltpu.CompilerParams(needs_layout_passes=False),
        scratch_types=[
            pltpu.VMEM((2 * L * D,), jnp.float32),
            pltpu.VMEM((2 * D,), jnp.float32),
            pltpu.VMEM((NCHUNK, CHUNK), jnp.int32),
            pltpu.VMEM((TPW,), jnp.int32),
            pltpu.VMEM((CHUNK, D), jnp.float32),
            pltpu.VMEM((CHUNK, D), jnp.float32),
            pltpu.SemaphoreType.DMA,
        ],
    )(ids, ps_idx, token_table, ps, gb)
    return out.reshape(B, L, D)
